# baseline (speedup 1.0000x reference)
import functools

import jax
import jax.numpy as jnp
from jax import lax
from jax.experimental import pallas as pl
from jax.experimental.pallas import tpu as pltpu
from jax.experimental.pallas import tpu_sc as plsc

N = 10000
D = 128
ND = 10240
DUMMY = 10048
B = 96
NTILES = 32
RPT = ND // 16


def _prep_body(x_ref, w_ref, att_ref, h_ref, a_ref):
    h = jnp.dot(x_ref[...], w_ref[...], preferred_element_type=jnp.float32)
    h_ref[...] = h
    a_ref[...] = jnp.dot(h, att_ref[...], preferred_element_type=jnp.float32)


def _fin_body(p_ref, d_ref, bias_ref, o_ref):
    num = p_ref[0] + p_ref[1]
    den = d_ref[0] + d_ref[1]
    o_ref[...] = num / (den + 1e-16) + bias_ref[...]


def _edge_body(nblk, asrc_hbm, adst_hbm, h_hbm, src_hbm, dst_hbm,
               feat_hbm, den0_hbm, den1_hbm,
               asrc_v, adst_v, si0, si1, si2, di0, di1, di2, w0, w1,
               r0, r1, zbuf, acc, den,
               gs0, gs1, ss0, ss1, is0, is1, is2):
    cid = lax.axis_index("c")
    sid = lax.axis_index("s")
    wid = sid * 2 + cid
    si = (si0, si1, si2)
    di = (di0, di1, di2)
    wb = (w0, w1)
    rows = (r0, r1)
    gsem = (gs0, gs1)
    ssem = (ss0, ss1)
    isem = (is0, is1, is2)

    pltpu.sync_copy(asrc_hbm, asrc_v)
    pltpu.sync_copy(adst_hbm, adst_v)

    def _zrow(j, carry):
        for r in range(D // 16):
            r0[j, pl.ds(r * 16, 16)] = jnp.zeros((16,), jnp.float32)
        return carry
    lax.fori_loop(0, B, _zrow, 0)
    for r in range(RPT // 16 + 1):
        zbuf[pl.ds(r * 16, 16)] = jnp.zeros((16,), jnp.float32)
    off = 0
    while off < RPT:
        nrow = min(B, RPT - off)
        pltpu.sync_copy(r0.at[pl.ds(0, nrow)],
                        acc.at[pl.ds(sid * RPT + off, nrow)])
        off += nrow
    pltpu.sync_copy(zbuf.at[pl.ds(0, RPT)], den.at[pl.ds(sid * RPT, RPT)])
    plsc.subcore_barrier()

    tile_base = wid * nblk * B

    def _issue_idx(b, s3):
        base = tile_base + b * B
        pltpu.async_copy(src_hbm.at[pl.ds(base, B)], si[s3], isem[s3])
        pltpu.async_copy(dst_hbm.at[pl.ds(base, B)], di[s3], isem[s3])

    def _wait_idx(s3):
        pltpu.make_async_copy(src_hbm.at[pl.ds(0, B)], si[s3],
                              isem[s3]).wait()
        pltpu.make_async_copy(dst_hbm.at[pl.ds(0, B)], di[s3],
                              isem[s3]).wait()

    def _compute_w(p2, s3):
        for i in range(B // 16):
            sids = si[s3][pl.ds(i * 16, 16)]
            dids = di[s3][pl.ds(i * 16, 16)]
            al = (plsc.load_gather(asrc_v, [sids])
                  + plsc.load_gather(adst_v, [dids]))
            al = jnp.where(al >= 0.0, al, 0.2 * al)
            wb[p2][pl.ds(i * 16, 16)] = jnp.exp(al)

    def _issue_scatter(p2, s3):
        pltpu.async_copy(rows[p2], acc.at[di[s3]], ssem[p2], add=True)
        pltpu.async_copy(wb[p2].at[pl.ds(0, B)], den.at[di[s3]], ssem[p2],
                         add=True)

    def _drain_scatter(p2, s3):
        pltpu.make_async_copy(rows[p2], acc.at[di[s3]], ssem[p2]).wait()
        pltpu.make_async_copy(wb[p2].at[pl.ds(0, B)], den.at[di[s3]],
                              ssem[p2]).wait()

    _issue_idx(0, 0)
    _issue_idx(1, 1)
    _wait_idx(0)
    pltpu.async_copy(h_hbm.at[si[0]], rows[0], gsem[0])

    nb6 = nblk // 6

    def _outer(b6, carry):
        for k in range(6):
            p2, q2, p3 = k % 2, 1 - k % 2, k % 3
            s_prev = (k + 2) % 3
            s_next = (k + 1) % 3
            b = b6 * 6 + k
            _compute_w(p2, p3)
            if k == 0:
                @pl.when(b6 > 0)
                def _():
                    _drain_scatter(q2, s_prev)
            else:
                _drain_scatter(q2, s_prev)
            if k < 4:
                _issue_idx(b + 2, s_prev)
            else:
                @pl.when(b6 < nb6 - 1)
                def _():
                    _issue_idx(b + 2, s_prev)
            if k < 5:
                _wait_idx(s_next)
                pltpu.async_copy(h_hbm.at[si[s_next]], rows[q2], gsem[q2])
            else:
                @pl.when(b6 < nb6 - 1)
                def _():
                    _wait_idx(s_next)
                    pltpu.async_copy(h_hbm.at[si[s_next]], rows[q2],
                                     gsem[q2])
            pltpu.make_async_copy(h_hbm.at[si[p3]], rows[p2],
                                  gsem[p2]).wait()

            @plsc.parallel_loop(0, B, unroll=8)
            def _(j):
                ws = wb[p2][pl.ds(j, 16)][0]
                for r in range(D // 16):
                    rows[p2][j, pl.ds(r * 16, 16)] = (
                        rows[p2][j, pl.ds(r * 16, 16)] * ws)

            _issue_scatter(p2, p3)
        return carry
    lax.fori_loop(0, nb6, _outer, 0)
    _drain_scatter(1, 2)

    plsc.subcore_barrier()
    pltpu.sync_copy(acc.at[pl.ds(sid * RPT, RPT)],
                    feat_hbm.at[cid, pl.ds(sid * RPT, RPT)])

    @pl.when(cid == 0)
    def _():
        pltpu.sync_copy(den.at[pl.ds(sid * RPT, RPT)],
                        den0_hbm.at[pl.ds(sid * RPT, RPT)])

    @pl.when(cid == 1)
    def _():
        pltpu.sync_copy(den.at[pl.ds(sid * RPT, RPT)],
                        den1_hbm.at[pl.ds(sid * RPT, RPT)])


def kernel(x, edge_index, W, att_src, att_dst, bias):
    n = x.shape[0]
    e = edge_index.shape[1]
    etot = e + n
    nblk = -(-etot // (NTILES * B))
    nblk += (-nblk) % 6
    ep = NTILES * nblk * B

    att2 = jnp.stack([att_src, att_dst], axis=1)
    grid = 10
    rb = n // grid
    h, a = pl.pallas_call(
        _prep_body,
        grid=(grid,),
        in_specs=[
            pl.BlockSpec((rb, D), lambda i: (i, 0)),
            pl.BlockSpec((D, D), lambda i: (0, 0)),
            pl.BlockSpec((D, 2), lambda i: (0, 0)),
        ],
        out_specs=[
            pl.BlockSpec((rb, D), lambda i: (i, 0)),
            pl.BlockSpec((rb, 2), lambda i: (i, 0)),
        ],
        out_shape=[
            jax.ShapeDtypeStruct((n, D), jnp.float32),
            jax.ShapeDtypeStruct((n, 2), jnp.float32),
        ],
    )(x, W, att2)

    asrc = jnp.pad(a[:, 0], (0, ND - n))
    adst = jnp.pad(a[:, 1], (0, ND - n))
    loops = jnp.arange(n, dtype=jnp.int32)
    src = jnp.concatenate(
        [edge_index[0], loops, jnp.zeros((ep - etot,), jnp.int32)])
    dst = jnp.concatenate(
        [edge_index[1], loops, jnp.full((ep - etot,), DUMMY, jnp.int32)])

    mesh = plsc.VectorSubcoreMesh(
        core_axis_name="c", subcore_axis_name="s", num_cores=2,
        num_subcores=16)
    feat, den0, den1 = pl.kernel(
        functools.partial(_edge_body, nblk),
        out_type=[
            jax.ShapeDtypeStruct((2, ND, D), jnp.float32),
            jax.ShapeDtypeStruct((ND,), jnp.float32),
            jax.ShapeDtypeStruct((ND,), jnp.float32),
        ],
        mesh=mesh,
        compiler_params=pltpu.CompilerParams(needs_layout_passes=False),
        scratch_types=[
            pltpu.VMEM((ND,), jnp.float32),
            pltpu.VMEM((ND,), jnp.float32),
            pltpu.VMEM((B,), jnp.int32),
            pltpu.VMEM((B,), jnp.int32),
            pltpu.VMEM((B,), jnp.int32),
            pltpu.VMEM((B,), jnp.int32),
            pltpu.VMEM((B,), jnp.int32),
            pltpu.VMEM((B,), jnp.int32),
            pltpu.VMEM((B + 16,), jnp.float32),
            pltpu.VMEM((B + 16,), jnp.float32),
            pltpu.VMEM((B, D), jnp.float32),
            pltpu.VMEM((B, D), jnp.float32),
            pltpu.VMEM((RPT + 16,), jnp.float32),
            pltpu.VMEM_SHARED((ND, D), jnp.float32),
            pltpu.VMEM_SHARED((ND,), jnp.float32),
            pltpu.SemaphoreType.DMA,
            pltpu.SemaphoreType.DMA,
            pltpu.SemaphoreType.DMA,
            pltpu.SemaphoreType.DMA,
            pltpu.SemaphoreType.DMA,
            pltpu.SemaphoreType.DMA,
            pltpu.SemaphoreType.DMA,
        ],
    )(asrc, adst, h, src, dst)

    out = pl.pallas_call(
        _fin_body,
        grid=(grid,),
        in_specs=[
            pl.BlockSpec((2, rb, D), lambda i: (0, i, 0)),
            pl.BlockSpec((2, rb, 1), lambda i: (0, i, 0)),
            pl.BlockSpec((1, D), lambda i: (0, 0)),
        ],
        out_specs=pl.BlockSpec((rb, D), lambda i: (i, 0)),
        out_shape=jax.ShapeDtypeStruct((n, D), jnp.float32),
    )(feat, jnp.stack([den0, den1]).reshape(2, ND, 1), bias.reshape(1, D))
    return out


# speedup vs baseline: 1.0690x; 1.0690x over previous
import functools

import jax
import jax.numpy as jnp
from jax import lax
from jax.experimental import pallas as pl
from jax.experimental.pallas import tpu as pltpu
from jax.experimental.pallas import tpu_sc as plsc

N = 10000
D = 128
ND = 10240
DUMMY = 10048
B = 96
NTILES = 32
RPT = ND // 16


def _prep_body(e_total, n, orows, x_ref, w_ref, as_ref, ad_ref, e0_ref,
               e1_ref, h_ref, a0_ref, a1_ref, s_ref, d_ref):
    h = jnp.dot(x_ref[...], w_ref[...], preferred_element_type=jnp.float32)
    h_ref[...] = h
    a0_ref[...] = jnp.dot(h, as_ref[...], preferred_element_type=jnp.float32)
    a1_ref[...] = jnp.dot(h, ad_ref[...], preferred_element_type=jnp.float32)
    g = pl.program_id(0)
    row = lax.broadcasted_iota(jnp.int32, (orows, 128), 0)
    col = lax.broadcasted_iota(jnp.int32, (orows, 128), 1)
    gidx = g * (orows * 128) + row * 128 + col
    lo = gidx - e_total
    s_ref[...] = jnp.where(gidx < e_total, e0_ref[...],
                           jnp.where(lo < n, lo, 0))
    d_ref[...] = jnp.where(gidx < e_total, e1_ref[...],
                           jnp.where(lo < n, lo, DUMMY))


def _fin_body(p_ref, d0_ref, d1_ref, bias_ref, o_ref):
    num = p_ref[0] + p_ref[1]
    den = d0_ref[...] + d1_ref[...]
    o_ref[...] = num / (den + 1e-16) + bias_ref[...]


def _edge_body(nblk, asrc_hbm, adst_hbm, h_hbm, src_hbm, dst_hbm,
               feat_hbm, den0_hbm, den1_hbm,
               asrc_v, adst_v, si0, si1, si2, di0, di1, di2, w0, w1,
               r0, r1, zbuf, acc, den,
               gs0, gs1, ss0, ss1, is0, is1, is2):
    cid = lax.axis_index("c")
    sid = lax.axis_index("s")
    wid = sid * 2 + cid
    si = (si0, si1, si2)
    di = (di0, di1, di2)
    wb = (w0, w1)
    rows = (r0, r1)
    gsem = (gs0, gs1)
    ssem = (ss0, ss1)
    isem = (is0, is1, is2)

    nsrc = asrc_hbm.shape[0]
    pltpu.sync_copy(asrc_hbm, asrc_v.at[pl.ds(0, nsrc)])
    pltpu.sync_copy(adst_hbm, adst_v.at[pl.ds(0, nsrc)])

    def _zrow(j, carry):
        for r in range(D // 16):
            r0[j, pl.ds(r * 16, 16)] = jnp.zeros((16,), jnp.float32)
        return carry
    lax.fori_loop(0, B, _zrow, 0)
    for r in range(RPT // 16 + 1):
        zbuf[pl.ds(r * 16, 16)] = jnp.zeros((16,), jnp.float32)
    off = 0
    while off < RPT:
        nrow = min(B, RPT - off)
        pltpu.sync_copy(r0.at[pl.ds(0, nrow)],
                        acc.at[pl.ds(sid * RPT + off, nrow)])
        off += nrow
    pltpu.sync_copy(zbuf.at[pl.ds(0, RPT)], den.at[pl.ds(sid * RPT, RPT)])
    plsc.subcore_barrier()

    tile_base = wid * nblk * B

    def _issue_idx(b, s3):
        base = tile_base + b * B
        pltpu.async_copy(src_hbm.at[pl.ds(base, B)], si[s3], isem[s3])
        pltpu.async_copy(dst_hbm.at[pl.ds(base, B)], di[s3], isem[s3])

    def _wait_idx(s3):
        pltpu.make_async_copy(src_hbm.at[pl.ds(0, B)], si[s3],
                              isem[s3]).wait()
        pltpu.make_async_copy(dst_hbm.at[pl.ds(0, B)], di[s3],
                              isem[s3]).wait()

    def _compute_w(p2, s3):
        for i in range(B // 16):
            sids = si[s3][pl.ds(i * 16, 16)]
            dids = di[s3][pl.ds(i * 16, 16)]
            al = (plsc.load_gather(asrc_v, [sids])
                  + plsc.load_gather(adst_v, [dids]))
            al = jnp.where(al >= 0.0, al, 0.2 * al)
            wb[p2][pl.ds(i * 16, 16)] = jnp.exp(al)

    def _issue_scatter(p2, s3):
        pltpu.async_copy(rows[p2], acc.at[di[s3]], ssem[p2], add=True)
        pltpu.async_copy(wb[p2].at[pl.ds(0, B)], den.at[di[s3]], ssem[p2],
                         add=True)

    def _drain_scatter(p2, s3):
        pltpu.make_async_copy(rows[p2], acc.at[di[s3]], ssem[p2]).wait()
        pltpu.make_async_copy(wb[p2].at[pl.ds(0, B)], den.at[di[s3]],
                              ssem[p2]).wait()

    _issue_idx(0, 0)
    _issue_idx(1, 1)
    _wait_idx(0)
    pltpu.async_copy(h_hbm.at[si[0]], rows[0], gsem[0])

    nb6 = nblk // 6

    def _outer(b6, carry):
        for k in range(6):
            p2, q2, p3 = k % 2, 1 - k % 2, k % 3
            s_prev = (k + 2) % 3
            s_next = (k + 1) % 3
            b = b6 * 6 + k
            _compute_w(p2, p3)
            if k == 0:
                @pl.when(b6 > 0)
                def _():
                    _drain_scatter(q2, s_prev)
            else:
                _drain_scatter(q2, s_prev)
            if k < 4:
                _issue_idx(b + 2, s_prev)
            else:
                @pl.when(b6 < nb6 - 1)
                def _():
                    _issue_idx(b + 2, s_prev)
            if k < 5:
                _wait_idx(s_next)
                pltpu.async_copy(h_hbm.at[si[s_next]], rows[q2], gsem[q2])
            else:
                @pl.when(b6 < nb6 - 1)
                def _():
                    _wait_idx(s_next)
                    pltpu.async_copy(h_hbm.at[si[s_next]], rows[q2],
                                     gsem[q2])
            pltpu.make_async_copy(h_hbm.at[si[p3]], rows[p2],
                                  gsem[p2]).wait()

            @plsc.parallel_loop(0, B, unroll=8)
            def _(j):
                ws = wb[p2][pl.ds(j, 16)][0]
                for r in range(D // 16):
                    rows[p2][j, pl.ds(r * 16, 16)] = (
                        rows[p2][j, pl.ds(r * 16, 16)] * ws)

            _issue_scatter(p2, p3)
        return carry
    lax.fori_loop(0, nb6, _outer, 0)
    _drain_scatter(1, 2)

    plsc.subcore_barrier()
    pltpu.sync_copy(acc.at[pl.ds(sid * RPT, RPT)],
                    feat_hbm.at[cid, pl.ds(sid * RPT, RPT)])

    @pl.when(cid == 0)
    def _():
        pltpu.sync_copy(den.at[pl.ds(sid * RPT, RPT)],
                        den0_hbm.at[pl.ds(sid * RPT, RPT)])

    @pl.when(cid == 1)
    def _():
        pltpu.sync_copy(den.at[pl.ds(sid * RPT, RPT)],
                        den1_hbm.at[pl.ds(sid * RPT, RPT)])


def kernel(x, edge_index, W, att_src, att_dst, bias):
    n = x.shape[0]
    e = edge_index.shape[1]
    etot = e + n
    nblk = -(-etot // (NTILES * B))
    nblk += (-nblk) % 6
    ep = NTILES * nblk * B

    gprep = 2
    rb = n // gprep
    orows = ep // (gprep * 128)
    erows = e // 128
    e2d0 = edge_index[0].reshape(erows, 128)
    e2d1 = edge_index[1].reshape(erows, 128)
    h, a0, a1, src2d, dst2d = pl.pallas_call(
        functools.partial(_prep_body, e, n, orows),
        grid=(gprep,),
        in_specs=[
            pl.BlockSpec((rb, D), lambda i: (i, 0)),
            pl.BlockSpec((D, D), lambda i: (0, 0)),
            pl.BlockSpec((D, 1), lambda i: (0, 0)),
            pl.BlockSpec((D, 1), lambda i: (0, 0)),
            pl.BlockSpec((orows, 128), lambda i: (i, 0)),
            pl.BlockSpec((orows, 128), lambda i: (i, 0)),
        ],
        out_specs=[
            pl.BlockSpec((rb, D), lambda i: (i, 0)),
            pl.BlockSpec((rb, 1), lambda i: (i, 0)),
            pl.BlockSpec((rb, 1), lambda i: (i, 0)),
            pl.BlockSpec((orows, 128), lambda i: (i, 0)),
            pl.BlockSpec((orows, 128), lambda i: (i, 0)),
        ],
        out_shape=[
            jax.ShapeDtypeStruct((n, D), jnp.float32),
            jax.ShapeDtypeStruct((n, 1), jnp.float32),
            jax.ShapeDtypeStruct((n, 1), jnp.float32),
            jax.ShapeDtypeStruct((ep // 128, 128), jnp.int32),
            jax.ShapeDtypeStruct((ep // 128, 128), jnp.int32),
        ],
    )(x, W, att_src.reshape(D, 1), att_dst.reshape(D, 1), e2d0, e2d1)
    asrc = a0.reshape(n)
    adst = a1.reshape(n)
    src = src2d.reshape(ep)
    dst = dst2d.reshape(ep)

    mesh = plsc.VectorSubcoreMesh(
        core_axis_name="c", subcore_axis_name="s", num_cores=2,
        num_subcores=16)
    feat, den0, den1 = pl.kernel(
        functools.partial(_edge_body, nblk),
        out_type=[
            jax.ShapeDtypeStruct((2, ND, D), jnp.float32),
            jax.ShapeDtypeStruct((ND,), jnp.float32),
            jax.ShapeDtypeStruct((ND,), jnp.float32),
        ],
        mesh=mesh,
        compiler_params=pltpu.CompilerParams(needs_layout_passes=False),
        scratch_types=[
            pltpu.VMEM((ND,), jnp.float32),
            pltpu.VMEM((ND,), jnp.float32),
            pltpu.VMEM((B,), jnp.int32),
            pltpu.VMEM((B,), jnp.int32),
            pltpu.VMEM((B,), jnp.int32),
            pltpu.VMEM((B,), jnp.int32),
            pltpu.VMEM((B,), jnp.int32),
            pltpu.VMEM((B,), jnp.int32),
            pltpu.VMEM((B + 16,), jnp.float32),
            pltpu.VMEM((B + 16,), jnp.float32),
            pltpu.VMEM((B, D), jnp.float32),
            pltpu.VMEM((B, D), jnp.float32),
            pltpu.VMEM((RPT + 16,), jnp.float32),
            pltpu.VMEM_SHARED((ND, D), jnp.float32),
            pltpu.VMEM_SHARED((ND,), jnp.float32),
            pltpu.SemaphoreType.DMA,
            pltpu.SemaphoreType.DMA,
            pltpu.SemaphoreType.DMA,
            pltpu.SemaphoreType.DMA,
            pltpu.SemaphoreType.DMA,
            pltpu.SemaphoreType.DMA,
            pltpu.SemaphoreType.DMA,
        ],
    )(asrc, adst, h, src, dst)

    gfin = 10
    fb = n // gfin
    out = pl.pallas_call(
        _fin_body,
        grid=(gfin,),
        in_specs=[
            pl.BlockSpec((2, fb, D), lambda i: (0, i, 0)),
            pl.BlockSpec((fb, 1), lambda i: (i, 0)),
            pl.BlockSpec((fb, 1), lambda i: (i, 0)),
            pl.BlockSpec((1, D), lambda i: (0, 0)),
        ],
        out_specs=pl.BlockSpec((fb, D), lambda i: (i, 0)),
        out_shape=jax.ShapeDtypeStruct((n, D), jnp.float32),
    )(feat, den0.reshape(ND, 1), den1.reshape(ND, 1), bias.reshape(1, D))
    return out
